# baseline (speedup 1.0000x reference)
import functools

import jax
import jax.numpy as jnp
from jax import lax
from jax.experimental import pallas as pl
from jax.experimental.pallas import tpu as pltpu
from jax.experimental.pallas import tpu_sc as plsc

N_NODES = 10000
N_PAD = 10240
NC = 2
NS = 16
NW = NC * NS
CHUNK = 128
KROW = 1
SUPER = KROW * CHUNK
NBUF = 4
SPAN = N_PAD // NS

_mesh = plsc.VectorSubcoreMesh(
    core_axis_name="c", subcore_axis_name="s", num_cores=NC, num_subcores=NS
)
_sc_params = pltpu.CompilerParams(use_tc_tiling_on_sc=False)


def _deg_body(nsup, dst_hbm, zeros_hbm, out_hbm, dst_v, ones_v, acc):
    cid = lax.axis_index("c")
    sid = lax.axis_index("s")
    wid = cid * NS + sid
    pltpu.sync_copy(zeros_hbm.at[pl.ds(sid * SPAN, SPAN)],
                    acc.at[pl.ds(sid * SPAN, SPAN)])
    for i in range(SUPER // 16):
        ones_v[pl.ds(i * 16, 16)] = jnp.full((16,), 1.0, jnp.float32)
    pltpu.sync_copy(dst_hbm.at[wid], dst_v)
    plsc.subcore_barrier()

    def sup_step(s, carry):
        pltpu.sync_copy(ones_v, acc.at[dst_v.at[s]],
                        add=True)
        return carry

    lax.fori_loop(0, nsup, sup_step, 0)
    plsc.subcore_barrier()
    pltpu.sync_copy(acc.at[pl.ds(sid * SPAN, SPAN)],
                    out_hbm.at[cid, pl.ds(sid * SPAN, SPAN)])


def _edge_body(nsup, table_hbm, src_hbm, dst_hbm, zeros_hbm, out_hbm,
               src_v, dst_v, rows0, rows1, acc, ssem0, ssem1):
    rows = (rows0, rows1)
    ssem = (ssem0, ssem1)
    cid = lax.axis_index("c")
    sid = lax.axis_index("s")
    wid = cid * NS + sid
    pltpu.sync_copy(zeros_hbm.at[pl.ds(sid * SPAN, SPAN)],
                    acc.at[pl.ds(sid * SPAN, SPAN)])
    pltpu.sync_copy(src_hbm.at[wid], src_v)
    pltpu.sync_copy(dst_hbm.at[wid], dst_v)
    plsc.subcore_barrier()

    for h in range(2):
        pltpu.sync_copy(table_hbm.at[src_v.at[h]], rows[h])
        pltpu.async_copy(rows[h], acc.at[dst_v.at[h]], ssem[h], add=True)

    def round_step(i, carry):
        for h in range(2):
            s = i * 2 + h
            pltpu.make_async_copy(rows[h], acc.at[dst_v.at[s - 2]],
                                  ssem[h]).wait()
            pltpu.sync_copy(table_hbm.at[src_v.at[s]], rows[h])
            pltpu.async_copy(rows[h], acc.at[dst_v.at[s]], ssem[h], add=True)
        return carry

    lax.fori_loop(1, nsup // 2, round_step, 0)
    for h in range(2):
        pltpu.make_async_copy(rows[h], acc.at[dst_v.at[h]], ssem[h]).wait()
    plsc.subcore_barrier()
    pltpu.sync_copy(acc.at[pl.ds(sid * SPAN, SPAN)],
                    out_hbm.at[cid, pl.ds(sid * SPAN, SPAN)])


def _tc1_body(x_ref, w1_ref, degp_ref, g1_ref, dinv_ref):
    deg = degp_ref[0] + degp_ref[1] + 1.0
    dinv = lax.rsqrt(deg)
    h = jnp.dot(x_ref[...], w1_ref[...], preferred_element_type=jnp.float32)
    g1_ref[...] = h * dinv
    dinv_ref[...] = dinv


def _tc2_body(p_ref, g1_ref, dinv_ref, b1_ref, g2_ref):
    agg = p_ref[0] + p_ref[1] + g1_ref[...]
    z = jnp.maximum(agg * dinv_ref[...] + b1_ref[...][None, :], 0.0)
    g2_ref[...] = z * dinv_ref[...]


def _tc3_body(p_ref, g2_ref, dinv_ref, w2_ref, b2_ref, out_ref):
    a = (p_ref[0] + p_ref[1] + g2_ref[...]) * dinv_ref[...]
    out_ref[...] = (
        jnp.dot(a, w2_ref[...], preferred_element_type=jnp.float32)
        + b2_ref[...][None, :]
    )


def kernel(x, edge_index, W1, b1, W2, b2):
    n = x.shape[0]
    e = edge_index.shape[1]
    nsup = -(-e // (NW * SUPER))
    nsup += nsup % 2
    nsup_alloc = nsup

    e_real = NW * SUPER * nsup
    ei = edge_index.astype(jnp.int32)
    pad = jnp.full((e_real - e,), n, jnp.int32)
    src3 = jnp.concatenate([ei[0], pad]).reshape(NW, nsup, SUPER)
    dst3 = jnp.concatenate([ei[1], pad]).reshape(NW, nsup, SUPER)

    x_pad = jnp.pad(x, ((0, N_PAD - n), (0, 0)))
    zeros1 = jnp.zeros((N_PAD,), jnp.float32)
    zeros16 = jnp.zeros((N_PAD, 16), jnp.float32)

    deg_call = pl.kernel(
        functools.partial(_deg_body, nsup),
        out_type=jax.ShapeDtypeStruct((NC, N_PAD), jnp.float32),
        mesh=_mesh,
        compiler_params=_sc_params,
        scratch_types=[
            pltpu.VMEM((nsup_alloc, SUPER), jnp.int32),
            pltpu.VMEM((SUPER,), jnp.float32),
            pltpu.VMEM_SHARED((N_PAD,), jnp.float32),
        ],
    )
    degp = deg_call(dst3, zeros1).reshape(NC, N_PAD, 1)


    edge_call = pl.kernel(
        functools.partial(_edge_body, nsup),
        out_type=jax.ShapeDtypeStruct((NC, N_PAD, 16), jnp.float32),
        mesh=_mesh,
        compiler_params=_sc_params,
        scratch_types=[
            pltpu.VMEM((nsup_alloc, SUPER), jnp.int32),
            pltpu.VMEM((nsup_alloc, SUPER), jnp.int32),
            pltpu.VMEM((SUPER, 16), jnp.float32),
            pltpu.VMEM((SUPER, 16), jnp.float32),
            pltpu.VMEM_SHARED((N_PAD, 16), jnp.float32),
            pltpu.SemaphoreType.DMA,
            pltpu.SemaphoreType.DMA,
        ],
    )

    g1, dinv = pl.pallas_call(
        _tc1_body,
        out_shape=(
            jax.ShapeDtypeStruct((N_PAD, 16), jnp.float32),
            jax.ShapeDtypeStruct((N_PAD, 1), jnp.float32),
        ),
    )(x_pad, W1, degp)

    p1 = edge_call(g1, src3, dst3, zeros16)

    g2 = pl.pallas_call(
        _tc2_body,
        out_shape=jax.ShapeDtypeStruct((N_PAD, 16), jnp.float32),
    )(p1, g1, dinv, b1)

    p2 = edge_call(g2, src3, dst3, zeros16)

    out = pl.pallas_call(
        _tc3_body,
        out_shape=jax.ShapeDtypeStruct((N_PAD, 64), jnp.float32),
    )(p2, g2, dinv, W2, b2)

    return out[:n]


# speedup vs baseline: 1.1618x; 1.1618x over previous
import functools

import jax
import jax.numpy as jnp
from jax import lax
from jax.experimental import pallas as pl
from jax.experimental.pallas import tpu as pltpu
from jax.experimental.pallas import tpu_sc as plsc

N_NODES = 10000
N_PAD = 10240
NC = 2
NS = 16
NW = NC * NS
CHUNK = 128
KROW = 1
SUPER = KROW * CHUNK
NBUF = 4
SPAN = N_PAD // NS

_mesh = plsc.VectorSubcoreMesh(
    core_axis_name="c", subcore_axis_name="s", num_cores=NC, num_subcores=NS
)
_sc_params = pltpu.CompilerParams(use_tc_tiling_on_sc=False)


def _deg_body(nsup, dst_hbm, zeros_hbm, out_hbm, dst_v, ones_v, acc):
    cid = lax.axis_index("c")
    sid = lax.axis_index("s")
    wid = cid * NS + sid
    pltpu.sync_copy(zeros_hbm.at[pl.ds(sid * SPAN, SPAN)],
                    acc.at[pl.ds(sid * SPAN, SPAN)])
    for i in range(SUPER // 16):
        ones_v[pl.ds(i * 16, 16)] = jnp.full((16,), 1.0, jnp.float32)
    pltpu.sync_copy(dst_hbm.at[wid], dst_v)
    plsc.subcore_barrier()

    def sup_step(s, carry):
        pltpu.sync_copy(ones_v, acc.at[dst_v.at[s]],
                        add=True)
        return carry

    lax.fori_loop(0, nsup, sup_step, 0)
    plsc.subcore_barrier()
    pltpu.sync_copy(acc.at[pl.ds(sid * SPAN, SPAN)],
                    out_hbm.at[cid, pl.ds(sid * SPAN, SPAN)])


def _edge_body(nsup, table_hbm, src_hbm, dst_hbm, zeros_hbm, out_hbm,
               src_v, dst_v, rows_v, acc):
    cid = lax.axis_index("c")
    sid = lax.axis_index("s")
    wid = cid * NS + sid
    pltpu.sync_copy(zeros_hbm.at[pl.ds(sid * SPAN, SPAN)],
                    acc.at[pl.ds(sid * SPAN, SPAN)])
    pltpu.sync_copy(src_hbm.at[wid], src_v)
    pltpu.sync_copy(dst_hbm.at[wid], dst_v)
    plsc.subcore_barrier()

    def sup_step(s, carry):
        pltpu.sync_copy(table_hbm.at[src_v.at[s]], rows_v)
        pltpu.sync_copy(rows_v, acc.at[dst_v.at[s]], add=True)
        return carry

    lax.fori_loop(0, nsup, sup_step, 0)
    plsc.subcore_barrier()
    pltpu.sync_copy(acc.at[pl.ds(sid * SPAN, SPAN)],
                    out_hbm.at[cid, pl.ds(sid * SPAN, SPAN)])


def _tc1_body(x_ref, w1_ref, degp_ref, g1_ref, dinv_ref):
    deg = degp_ref[0] + degp_ref[1] + 1.0
    dinv16 = jnp.broadcast_to(lax.rsqrt(deg)[:, None], g1_ref.shape)
    h = jnp.dot(x_ref[...], w1_ref[...], preferred_element_type=jnp.float32)
    g1_ref[...] = h * dinv16
    dinv_ref[...] = dinv16


def _tc2_body(p_ref, g1_ref, dinv_ref, b1_ref, g2_ref):
    agg = p_ref[0] + p_ref[1] + g1_ref[...]
    z = jnp.maximum(agg * dinv_ref[...] + b1_ref[...][None, :], 0.0)
    g2_ref[...] = z * dinv_ref[...]


def _tc3_body(p_ref, g2_ref, dinv_ref, w2_ref, b2_ref, out_ref):
    a = (p_ref[0] + p_ref[1] + g2_ref[...]) * dinv_ref[...]
    out_ref[...] = (
        jnp.dot(a, w2_ref[...], preferred_element_type=jnp.float32)
        + b2_ref[...][None, :]
    )


def kernel(x, edge_index, W1, b1, W2, b2):
    n = x.shape[0]
    e = edge_index.shape[1]
    nsup = -(-e // (NW * SUPER))
    nsup_alloc = nsup

    e_real = NW * SUPER * nsup
    ei = edge_index.astype(jnp.int32)
    pad = jnp.full((e_real - e,), n, jnp.int32)
    src3 = jnp.concatenate([ei[0], pad]).reshape(NW, nsup, SUPER)
    dst3 = jnp.concatenate([ei[1], pad]).reshape(NW, nsup, SUPER)

    x_pad = jnp.pad(x, ((0, N_PAD - n), (0, 0)))
    zeros1 = jnp.zeros((N_PAD,), jnp.float32)
    zeros16 = jnp.zeros((N_PAD, 16), jnp.float32)

    deg_call = pl.kernel(
        functools.partial(_deg_body, nsup),
        out_type=jax.ShapeDtypeStruct((NC, N_PAD), jnp.float32),
        mesh=_mesh,
        compiler_params=_sc_params,
        scratch_types=[
            pltpu.VMEM((nsup_alloc, SUPER), jnp.int32),
            pltpu.VMEM((SUPER,), jnp.float32),
            pltpu.VMEM_SHARED((N_PAD,), jnp.float32),
        ],
    )
    degp = deg_call(dst3, zeros1)


    edge_call = pl.kernel(
        functools.partial(_edge_body, nsup),
        out_type=jax.ShapeDtypeStruct((NC, N_PAD, 16), jnp.float32),
        mesh=_mesh,
        compiler_params=_sc_params,
        scratch_types=[
            pltpu.VMEM((nsup_alloc, SUPER), jnp.int32),
            pltpu.VMEM((nsup_alloc, SUPER), jnp.int32),
            pltpu.VMEM((SUPER, 16), jnp.float32),
            pltpu.VMEM_SHARED((N_PAD, 16), jnp.float32),
        ],
    )

    g1, dinv = pl.pallas_call(
        _tc1_body,
        out_shape=(
            jax.ShapeDtypeStruct((N_PAD, 16), jnp.float32),
            jax.ShapeDtypeStruct((N_PAD, 16), jnp.float32),
        ),
    )(x_pad, W1, degp)

    p1 = edge_call(g1, src3, dst3, zeros16)

    g2 = pl.pallas_call(
        _tc2_body,
        out_shape=jax.ShapeDtypeStruct((N_PAD, 16), jnp.float32),
    )(p1, g1, dinv, b1)

    p2 = edge_call(g2, src3, dst3, zeros16)

    out = pl.pallas_call(
        _tc3_body,
        out_shape=jax.ShapeDtypeStruct((N_PAD, 64), jnp.float32),
    )(p2, g2, dinv, W2, b2)

    return out[:n]
